# Initial kernel scaffold; baseline (speedup 1.0000x reference)
import jax
import jax.numpy as jnp
from jax.experimental import pallas as pl


def kernel(x, router_weights, scale, gate_kernel, gate_bias, up_kernel, up_bias, out_kernel, out_bias):
    raise NotImplementedError("write your pallas kernel here")


# speedup vs baseline: 1.6647x; 1.6647x over previous
import functools

import jax
import jax.numpy as jnp
from jax.experimental import pallas as pl

B, S, D, H, E = 1, 2048, 1024, 2816, 8
TS = 512
_STEPS = S // TS


def _body(x_ref, wr_ref, scale_ref, wg_ref, bg_ref, wu_ref, bu_ref,
          wo_ref, bo_ref, out_ref, usage_ref, loss_ref):
    i = pl.program_id(0)
    xt = x_ref[...]
    m = jnp.mean(xt * xt, axis=1, keepdims=True)

    logits = jnp.dot(xt, wr_ref[...],
                     preferred_element_type=jnp.float32,
                     precision=jax.lax.Precision.HIGHEST)
    mx = jnp.max(logits, axis=1, keepdims=True)
    ex = jnp.exp(logits - mx)
    probs = ex / jnp.sum(ex, axis=1, keepdims=True)

    w1 = jnp.max(probs, axis=1, keepdims=True)
    idx = jax.lax.broadcasted_iota(jnp.int32, probs.shape, 1)
    i1 = jnp.min(jnp.where(probs == w1, idx, E), axis=1, keepdims=True)
    w2 = jnp.max(jnp.where(idx == i1, -1.0, probs), axis=1, keepdims=True)
    s = w1 + w2
    w1n = w1 / s
    w2n = w2 / s
    c1 = w1n * jax.lax.rsqrt(w1n * w1n * m + 1e-6)
    c2 = w2n * jax.lax.rsqrt(w2n * w2n * m + 1e-6)

    xs = (xt * scale_ref[...]).astype(jnp.bfloat16)
    z = jnp.dot(xs, wg_ref[...], preferred_element_type=jnp.float32)
    v = jnp.dot(xs, wu_ref[...], preferred_element_type=jnp.float32)
    bg = bg_ref[...]
    bu = bu_ref[...]

    def act(c):
        g = c * z + bg
        return g * jax.nn.sigmoid(g) * (c * v + bu)

    comb = (act(c1) + act(c2)).astype(jnp.bfloat16)
    out = jnp.dot(comb, wo_ref[...], preferred_element_type=jnp.float32)
    out_ref[...] = out + 2.0 * bo_ref[...]

    ps = jnp.sum(probs, axis=0, keepdims=True)

    @pl.when(i == 0)
    def _():
        usage_ref[...] = ps

    @pl.when(i > 0)
    def _():
        usage_ref[...] += ps

    @pl.when(i == _STEPS - 1)
    def _():
        eu = usage_ref[...] / (B * S)
        loss_ref[...] = -jnp.sum(eu * jnp.log(eu + 1e-6)).reshape(1, 1)


@functools.partial(jax.jit, static_argnames=())
def kernel(x, router_weights, scale, gate_kernel, gate_bias, up_kernel,
           up_bias, out_kernel, out_bias):
    x2 = x.reshape(S, D)
    wg = gate_kernel.astype(jnp.bfloat16)
    wu = up_kernel.astype(jnp.bfloat16)
    wo = out_kernel.astype(jnp.bfloat16)
    scale2 = scale.reshape(1, D)
    bg2 = gate_bias.reshape(1, H)
    bu2 = up_bias.reshape(1, H)
    bo2 = out_bias.reshape(1, D)

    const = lambda shape: pl.BlockSpec(shape, lambda i: (0, 0))
    out, usage_sum, loss = pl.pallas_call(
        _body,
        grid=(_STEPS,),
        in_specs=[
            pl.BlockSpec((TS, D), lambda i: (i, 0)),
            const((D, E)),
            const((1, D)),
            const((D, H)),
            const((1, H)),
            const((D, H)),
            const((1, H)),
            const((H, D)),
            const((1, D)),
        ],
        out_specs=[
            pl.BlockSpec((TS, D), lambda i: (i, 0)),
            const((1, E)),
            const((1, 1)),
        ],
        out_shape=[
            jax.ShapeDtypeStruct((S, D), jnp.float32),
            jax.ShapeDtypeStruct((1, E), jnp.float32),
            jax.ShapeDtypeStruct((1, 1), jnp.float32),
        ],
    )(x2, router_weights, scale2, wg, bg2, wu, bu2, wo, bo2)
    return out.reshape(B, S, D), loss.reshape(())


# speedup vs baseline: 1.9249x; 1.1563x over previous
import functools

import jax
import jax.numpy as jnp
from jax.experimental import pallas as pl

B, S, D, H, E = 1, 2048, 1024, 2816, 8
TS = 256
_STEPS = S // TS


def _body(x_ref, wr_ref, scale_ref, wg_ref, bg_ref, wu_ref, bu_ref,
          wo_ref, bo_ref, out_ref, usage_ref, loss_ref):
    i = pl.program_id(0)
    xt = x_ref[...]
    m = jnp.mean(xt * xt, axis=1, keepdims=True)

    logits = jnp.dot(xt, wr_ref[...],
                     preferred_element_type=jnp.float32,
                     precision=jax.lax.Precision.HIGHEST)
    mx = jnp.max(logits, axis=1, keepdims=True)
    ex = jnp.exp(logits - mx)
    probs = ex / jnp.sum(ex, axis=1, keepdims=True)

    w1 = jnp.max(probs, axis=1, keepdims=True)
    idx = jax.lax.broadcasted_iota(jnp.int32, probs.shape, 1)
    i1 = jnp.min(jnp.where(probs == w1, idx, E), axis=1, keepdims=True)
    w2 = jnp.max(jnp.where(idx == i1, -1.0, probs), axis=1, keepdims=True)
    s = w1 + w2
    w1n = w1 / s
    w2n = w2 / s
    c1 = w1n * jax.lax.rsqrt(w1n * w1n * m + 1e-6)
    c2 = w2n * jax.lax.rsqrt(w2n * w2n * m + 1e-6)

    xs = (xt * scale_ref[...]).astype(jnp.bfloat16)
    z = jnp.dot(xs, wg_ref[...].astype(jnp.bfloat16),
                preferred_element_type=jnp.float32)
    v = jnp.dot(xs, wu_ref[...].astype(jnp.bfloat16),
                preferred_element_type=jnp.float32)
    bg = bg_ref[...]
    bu = bu_ref[...]

    def act(c):
        g = c * z + bg
        return g * jax.nn.sigmoid(g) * (c * v + bu)

    comb = (act(c1) + act(c2)).astype(jnp.bfloat16)
    out = jnp.dot(comb, wo_ref[...].astype(jnp.bfloat16),
                  preferred_element_type=jnp.float32)
    out_ref[...] = out + 2.0 * bo_ref[...]

    ps = jnp.sum(probs, axis=0, keepdims=True)

    @pl.when(i == 0)
    def _():
        usage_ref[...] = ps

    @pl.when(i > 0)
    def _():
        usage_ref[...] += ps

    @pl.when(i == _STEPS - 1)
    def _():
        eu = usage_ref[...] / (B * S)
        loss_ref[...] = -jnp.sum(eu * jnp.log(eu + 1e-6)).reshape(1, 1)


@functools.partial(jax.jit, static_argnames=())
def kernel(x, router_weights, scale, gate_kernel, gate_bias, up_kernel,
           up_bias, out_kernel, out_bias):
    x2 = x.reshape(S, D)
    wg = gate_kernel
    wu = up_kernel
    wo = out_kernel
    scale2 = scale.reshape(1, D)
    bg2 = gate_bias.reshape(1, H)
    bu2 = up_bias.reshape(1, H)
    bo2 = out_bias.reshape(1, D)

    const = lambda shape: pl.BlockSpec(shape, lambda i: (0, 0))
    out, usage_sum, loss = pl.pallas_call(
        _body,
        grid=(_STEPS,),
        in_specs=[
            pl.BlockSpec((TS, D), lambda i: (i, 0)),
            const((D, E)),
            const((1, D)),
            const((D, H)),
            const((1, H)),
            const((D, H)),
            const((1, H)),
            const((H, D)),
            const((1, D)),
        ],
        out_specs=[
            pl.BlockSpec((TS, D), lambda i: (i, 0)),
            const((1, E)),
            const((1, 1)),
        ],
        out_shape=[
            jax.ShapeDtypeStruct((S, D), jnp.float32),
            jax.ShapeDtypeStruct((1, E), jnp.float32),
            jax.ShapeDtypeStruct((1, 1), jnp.float32),
        ],
    )(x2, router_weights, scale2, wg, bg2, wu, bu2, wo, bo2)
    return out.reshape(B, S, D), loss.reshape(())


# speedup vs baseline: 2.1722x; 1.1285x over previous
import functools

import jax
import jax.numpy as jnp
from jax.experimental import pallas as pl

B, S, D, H, E = 1, 2048, 1024, 2816, 8
TS = 256
_STEPS = S // TS


def _body(x_ref, wr_ref, scale_ref, wg_ref, bg_ref, wu_ref, bu_ref,
          wo_ref, bo_ref, out_ref, usage_ref, loss_ref):
    i = pl.program_id(0)
    xt = x_ref[...]
    m = jnp.mean(xt * xt, axis=1, keepdims=True)

    logits = jnp.dot(xt.astype(jnp.bfloat16),
                     wr_ref[...].astype(jnp.bfloat16),
                     preferred_element_type=jnp.float32)
    mx = jnp.max(logits, axis=1, keepdims=True)
    ex = jnp.exp(logits - mx)
    probs = ex / jnp.sum(ex, axis=1, keepdims=True)

    w1 = jnp.max(probs, axis=1, keepdims=True)
    idx = jax.lax.broadcasted_iota(jnp.int32, probs.shape, 1)
    i1 = jnp.min(jnp.where(probs == w1, idx, E), axis=1, keepdims=True)
    w2 = jnp.max(jnp.where(idx == i1, -1.0, probs), axis=1, keepdims=True)
    s = w1 + w2
    w1n = w1 / s
    w2n = w2 / s
    c1 = w1n * jax.lax.rsqrt(w1n * w1n * m + 1e-6)
    c2 = w2n * jax.lax.rsqrt(w2n * w2n * m + 1e-6)

    xs = (xt * scale_ref[...]).astype(jnp.bfloat16)
    z = jnp.dot(xs, wg_ref[...].astype(jnp.bfloat16),
                preferred_element_type=jnp.float32)
    v = jnp.dot(xs, wu_ref[...].astype(jnp.bfloat16),
                preferred_element_type=jnp.float32)
    bg = bg_ref[...]
    bu = bu_ref[...]

    def act(c):
        g = c * z + bg
        return g * jax.nn.sigmoid(g) * (c * v + bu)

    comb = (act(c1) + act(c2)).astype(jnp.bfloat16)
    out = jnp.dot(comb, wo_ref[...].astype(jnp.bfloat16),
                  preferred_element_type=jnp.float32)
    out_ref[...] = out + 2.0 * bo_ref[...]

    ps = jnp.sum(probs, axis=0, keepdims=True)

    @pl.when(i == 0)
    def _():
        usage_ref[...] = ps

    @pl.when(i > 0)
    def _():
        usage_ref[...] += ps

    @pl.when(i == _STEPS - 1)
    def _():
        eu = usage_ref[...] / (B * S)
        loss_ref[...] = -jnp.sum(eu * jnp.log(eu + 1e-6)).reshape(1, 1)


@functools.partial(jax.jit, static_argnames=())
def kernel(x, router_weights, scale, gate_kernel, gate_bias, up_kernel,
           up_bias, out_kernel, out_bias):
    x2 = x.reshape(S, D)
    wg = gate_kernel
    wu = up_kernel
    wo = out_kernel
    scale2 = scale.reshape(1, D)
    bg2 = gate_bias.reshape(1, H)
    bu2 = up_bias.reshape(1, H)
    bo2 = out_bias.reshape(1, D)

    const = lambda shape: pl.BlockSpec(shape, lambda i: (0, 0))
    out, usage_sum, loss = pl.pallas_call(
        _body,
        grid=(_STEPS,),
        in_specs=[
            pl.BlockSpec((TS, D), lambda i: (i, 0)),
            const((D, E)),
            const((1, D)),
            const((D, H)),
            const((1, H)),
            const((D, H)),
            const((1, H)),
            const((H, D)),
            const((1, D)),
        ],
        out_specs=[
            pl.BlockSpec((TS, D), lambda i: (i, 0)),
            const((1, E)),
            const((1, 1)),
        ],
        out_shape=[
            jax.ShapeDtypeStruct((S, D), jnp.float32),
            jax.ShapeDtypeStruct((1, E), jnp.float32),
            jax.ShapeDtypeStruct((1, 1), jnp.float32),
        ],
    )(x2, router_weights, scale2, wg, bg2, wu, bu2, wo, bo2)
    return out.reshape(B, S, D), loss.reshape(())


# speedup vs baseline: 2.2150x; 1.0197x over previous
import functools

import jax
import jax.numpy as jnp
from jax.experimental import pallas as pl

B, S, D, H, E = 1, 2048, 1024, 2816, 8
TS = 512
_STEPS = S // TS


def _body(x_ref, wr_ref, scale_ref, wg_ref, bg_ref, wu_ref, bu_ref,
          wo_ref, bo_ref, out_ref, usage_ref, loss_ref):
    i = pl.program_id(0)
    xt = x_ref[...]
    m = jnp.mean(xt * xt, axis=1, keepdims=True)

    logits = jnp.dot(xt.astype(jnp.bfloat16),
                     wr_ref[...].astype(jnp.bfloat16),
                     preferred_element_type=jnp.float32)
    mx = jnp.max(logits, axis=1, keepdims=True)
    ex = jnp.exp(logits - mx)
    probs = ex / jnp.sum(ex, axis=1, keepdims=True)

    w1 = jnp.max(probs, axis=1, keepdims=True)
    idx = jax.lax.broadcasted_iota(jnp.int32, probs.shape, 1)
    i1 = jnp.min(jnp.where(probs == w1, idx, E), axis=1, keepdims=True)
    w2 = jnp.max(jnp.where(idx == i1, -1.0, probs), axis=1, keepdims=True)
    s = w1 + w2
    w1n = w1 / s
    w2n = w2 / s
    c1 = w1n * jax.lax.rsqrt(w1n * w1n * m + 1e-6)
    c2 = w2n * jax.lax.rsqrt(w2n * w2n * m + 1e-6)

    xs = (xt * scale_ref[...]).astype(jnp.bfloat16)
    z = jnp.dot(xs, wg_ref[...].astype(jnp.bfloat16),
                preferred_element_type=jnp.float32)
    v = jnp.dot(xs, wu_ref[...].astype(jnp.bfloat16),
                preferred_element_type=jnp.float32)
    bg = bg_ref[...]
    bu = bu_ref[...]

    def act(c):
        g = c * z + bg
        return g * jax.nn.sigmoid(g) * (c * v + bu)

    comb = (act(c1) + act(c2)).astype(jnp.bfloat16)
    out = jnp.dot(comb, wo_ref[...].astype(jnp.bfloat16),
                  preferred_element_type=jnp.float32)
    out_ref[...] = out + 2.0 * bo_ref[...]

    ps = jnp.sum(probs, axis=0, keepdims=True)

    @pl.when(i == 0)
    def _():
        usage_ref[...] = ps

    @pl.when(i > 0)
    def _():
        usage_ref[...] += ps

    @pl.when(i == _STEPS - 1)
    def _():
        eu = usage_ref[...] / (B * S)
        loss_ref[...] = -jnp.sum(eu * jnp.log(eu + 1e-6)).reshape(1, 1)


@functools.partial(jax.jit, static_argnames=())
def kernel(x, router_weights, scale, gate_kernel, gate_bias, up_kernel,
           up_bias, out_kernel, out_bias):
    x2 = x.reshape(S, D)
    wg = gate_kernel
    wu = up_kernel
    wo = out_kernel
    scale2 = scale.reshape(1, D)
    bg2 = gate_bias.reshape(1, H)
    bu2 = up_bias.reshape(1, H)
    bo2 = out_bias.reshape(1, D)

    const = lambda shape: pl.BlockSpec(shape, lambda i: (0, 0))
    out, usage_sum, loss = pl.pallas_call(
        _body,
        grid=(_STEPS,),
        in_specs=[
            pl.BlockSpec((TS, D), lambda i: (i, 0)),
            const((D, E)),
            const((1, D)),
            const((D, H)),
            const((1, H)),
            const((D, H)),
            const((1, H)),
            const((H, D)),
            const((1, D)),
        ],
        out_specs=[
            pl.BlockSpec((TS, D), lambda i: (i, 0)),
            const((1, E)),
            const((1, 1)),
        ],
        out_shape=[
            jax.ShapeDtypeStruct((S, D), jnp.float32),
            jax.ShapeDtypeStruct((1, E), jnp.float32),
            jax.ShapeDtypeStruct((1, 1), jnp.float32),
        ],
    )(x2, router_weights, scale2, wg, bg2, wu, bu2, wo, bo2)
    return out.reshape(B, S, D), loss.reshape(())
